# baseline (device time: 38512 ns/iter reference)
import jax
import jax.numpy as jnp
from jax import lax
from jax.experimental import pallas as pl
from jax.experimental.pallas import tpu as pltpu

N_PARTS = 4


def kernel(x, Win0, Wout0, Win1, Wout1, Win2, Wout2):
    b, d_half = x.shape
    h_half = Win0.shape[1]
    bf16 = jnp.bfloat16

    def body(x_ref, win0_ref, wout0_ref, win1_ref, wout1_ref, win2_ref,
             wout2_ref, out_ref,
             win_buf, wout_buf, h_send, h_recv, x_send, x_recv,
             w_sems, h_send_sems, h_recv_sems, x_send_sems, x_recv_sems):
        my_x = lax.axis_index("x")
        my_y = lax.axis_index("y")
        y_peer = (my_x, 1 - my_y)
        x_peer = (1 - my_x, my_y)

        win_hbm = (win0_ref, win1_ref, win2_ref)
        wout_hbm = (wout0_ref, wout1_ref, wout2_ref)

        def w_parts(seq, hbm_ref, buf, slot, nrows):
            q = nrows // N_PARTS
            bank = (seq % 2) * N_PARTS
            return [
                pltpu.make_async_copy(
                    hbm_ref.at[pl.ds(c * q, q)],
                    buf.at[slot, pl.ds(c * q, q)],
                    w_sems.at[bank + c],
                )
                for c in range(N_PARTS)
            ]

        def win_parts(layer):
            return w_parts(2 * layer, win_hbm[layer], win_buf,
                           layer % 2, d_half)

        def wout_parts(layer):
            return w_parts(2 * layer + 1, wout_hbm[layer], wout_buf,
                           layer % 2, h_half)

        def start_all(parts):
            for p in parts:
                p.start()

        def wait_all(parts):
            for p in parts:
                p.wait()

        start_all(win_parts(0))

        barrier = pltpu.get_barrier_semaphore()
        for peer in (y_peer, x_peer):
            pl.semaphore_signal(
                barrier, inc=1, device_id=peer,
                device_id_type=pl.DeviceIdType.MESH,
            )
        pl.semaphore_wait(barrier, 2)

        H2 = h_half // 2
        D2 = d_half // 2

        def start_chunk(layer, c, chunk_f32, send_buf, recv_buf, send_sems,
                        recv_sems, width, peer):
            cols = pl.ds(c * width, width)
            send_buf[layer, :, cols] = chunk_f32.astype(bf16)
            rdma = pltpu.make_async_remote_copy(
                src_ref=send_buf.at[layer, :, cols],
                dst_ref=recv_buf.at[layer, :, cols],
                send_sem=send_sems.at[2 * layer + c],
                recv_sem=recv_sems.at[2 * layer + c],
                device_id=peer,
                device_id_type=pl.DeviceIdType.MESH,
            )
            rdma.start()
            return rdma

        acts = x_ref[...]
        wait_all(win_parts(0))
        start_all(wout_parts(0))
        for layer in range(3):
            win = win_buf[layer % 2]
            wout = wout_buf[layer % 2]
            h_rdmas = []
            hps = []
            for c in range(2):
                hp_c = jnp.dot(acts, win[:, c * H2:(c + 1) * H2],
                               preferred_element_type=jnp.float32)
                h_rdmas.append(start_chunk(layer, c, hp_c, h_send, h_recv,
                                           h_send_sems, h_recv_sems, H2,
                                           y_peer))
                hps.append(hp_c)
            wait_all(wout_parts(layer))
            if layer < 2:
                start_all(win_parts(layer + 1))
            xp = None
            for c in range(2):
                h_rdmas[c].wait()
                cols = pl.ds(c * H2, H2)
                h_c = hps[c] + h_recv[layer, :, cols].astype(jnp.float32)
                h_c = jnp.maximum(h_c, 0.0)
                t = jnp.dot(h_c, wout[c * H2:(c + 1) * H2, :],
                            preferred_element_type=jnp.float32)
                xp = t if xp is None else xp + t
            x_rdmas = [
                start_chunk(layer, c, xp[:, c * D2:(c + 1) * D2], x_send,
                            x_recv, x_send_sems, x_recv_sems, D2, x_peer)
                for c in range(2)
            ]
            if layer < 2:
                wait_all(win_parts(layer + 1))
                start_all(wout_parts(layer + 1))
            for c in range(2):
                x_rdmas[c].wait()
            acts = xp + x_recv[layer].astype(jnp.float32)
        out_ref[...] = acts

    vmem = pl.BlockSpec(memory_space=pltpu.VMEM)
    hbm = pl.BlockSpec(memory_space=pl.ANY)
    return pl.pallas_call(
        body,
        out_shape=jax.ShapeDtypeStruct((b, d_half), jnp.float32),
        in_specs=[vmem] + [hbm] * 6,
        out_specs=vmem,
        scratch_shapes=[
            pltpu.VMEM((2, d_half, h_half), jnp.float32),
            pltpu.VMEM((2, h_half, d_half), jnp.float32),
            pltpu.VMEM((3, b, h_half), bf16),
            pltpu.VMEM((3, b, h_half), bf16),
            pltpu.VMEM((3, b, d_half), bf16),
            pltpu.VMEM((3, b, d_half), bf16),
            pltpu.SemaphoreType.DMA((2 * N_PARTS,)),
            pltpu.SemaphoreType.DMA((6,)),
            pltpu.SemaphoreType.DMA((6,)),
            pltpu.SemaphoreType.DMA((6,)),
            pltpu.SemaphoreType.DMA((6,)),
        ],
        compiler_params=pltpu.CompilerParams(
            collective_id=0, vmem_limit_bytes=100 * 1024 * 1024,
        ),
    )(x, Win0, Wout0, Win1, Wout1, Win2, Wout2)


# device time: 38492 ns/iter; 1.0005x vs baseline; 1.0005x over previous
import jax
import jax.numpy as jnp
from jax import lax
from jax.experimental import pallas as pl
from jax.experimental.pallas import tpu as pltpu

N_PARTS = 4


def kernel(x, Win0, Wout0, Win1, Wout1, Win2, Wout2):
    b, d_half = x.shape
    h_half = Win0.shape[1]
    bf16 = jnp.bfloat16

    def body(x_ref, win0_ref, wout0_ref, win1_ref, wout1_ref, win2_ref,
             wout2_ref, out_ref,
             win_buf, wout_buf, h_send, h_recv, x_send, x_recv,
             w_sems, h_send_sems, h_recv_sems, x_send_sems, x_recv_sems):
        my_x = lax.axis_index("x")
        my_y = lax.axis_index("y")
        y_peer = (my_x, 1 - my_y)
        x_peer = (1 - my_x, my_y)

        win_hbm = (win0_ref, win1_ref, win2_ref)
        wout_hbm = (wout0_ref, wout1_ref, wout2_ref)

        def w_parts(seq, hbm_ref, buf, slot, nrows):
            q = nrows // N_PARTS
            bank = (seq % 2) * N_PARTS
            return [
                pltpu.make_async_copy(
                    hbm_ref.at[pl.ds(c * q, q)],
                    buf.at[slot, pl.ds(c * q, q)],
                    w_sems.at[bank + c],
                )
                for c in range(N_PARTS)
            ]

        def win_parts(layer):
            return w_parts(2 * layer, win_hbm[layer], win_buf,
                           layer % 2, d_half)

        def wout_parts(layer):
            return w_parts(2 * layer + 1, wout_hbm[layer], wout_buf,
                           layer % 2, h_half)

        def start_all(parts):
            for p in parts:
                p.start()

        def wait_all(parts):
            for p in parts:
                p.wait()

        start_all(win_parts(0))

        barrier = pltpu.get_barrier_semaphore()
        for peer in (y_peer, x_peer):
            pl.semaphore_signal(
                barrier, inc=1, device_id=peer,
                device_id_type=pl.DeviceIdType.MESH,
            )
        pl.semaphore_wait(barrier, 2)

        H2 = h_half // 2
        D2 = d_half // 2

        def start_chunk(layer, c, chunk_f32, send_buf, recv_buf, send_sems,
                        recv_sems, peer):
            send_buf[layer, c] = chunk_f32.astype(bf16)
            rdma = pltpu.make_async_remote_copy(
                src_ref=send_buf.at[layer, c],
                dst_ref=recv_buf.at[layer, c],
                send_sem=send_sems.at[2 * layer + c],
                recv_sem=recv_sems.at[2 * layer + c],
                device_id=peer,
                device_id_type=pl.DeviceIdType.MESH,
            )
            rdma.start()
            return rdma

        acts = x_ref[...]
        wait_all(win_parts(0))
        start_all(wout_parts(0))
        for layer in range(3):
            win = win_buf[layer % 2]
            wout = wout_buf[layer % 2]
            h_rdmas = []
            hps = []
            for c in range(2):
                hp_c = jnp.dot(acts, win[:, c * H2:(c + 1) * H2],
                               preferred_element_type=jnp.float32)
                h_rdmas.append(start_chunk(layer, c, hp_c, h_send, h_recv,
                                           h_send_sems, h_recv_sems,
                                           y_peer))
                hps.append(hp_c)
            wait_all(wout_parts(layer))
            if layer < 2:
                start_all(win_parts(layer + 1))
            xp = None
            for c in range(2):
                h_rdmas[c].wait()
                h_c = hps[c] + h_recv[layer, c].astype(jnp.float32)
                h_c = jnp.maximum(h_c, 0.0)
                t = jnp.dot(h_c, wout[c * H2:(c + 1) * H2, :],
                            preferred_element_type=jnp.float32)
                xp = t if xp is None else xp + t
            x_rdmas = [
                start_chunk(layer, c, xp[:, c * D2:(c + 1) * D2], x_send,
                            x_recv, x_send_sems, x_recv_sems, x_peer)
                for c in range(2)
            ]
            if layer < 2:
                wait_all(win_parts(layer + 1))
                start_all(wout_parts(layer + 1))
            for c in range(2):
                x_rdmas[c].wait()
            acts = xp + jnp.concatenate(
                [x_recv[layer, 0], x_recv[layer, 1]], axis=1
            ).astype(jnp.float32)
        out_ref[...] = acts

    vmem = pl.BlockSpec(memory_space=pltpu.VMEM)
    hbm = pl.BlockSpec(memory_space=pl.ANY)
    return pl.pallas_call(
        body,
        out_shape=jax.ShapeDtypeStruct((b, d_half), jnp.float32),
        in_specs=[vmem] + [hbm] * 6,
        out_specs=vmem,
        scratch_shapes=[
            pltpu.VMEM((2, d_half, h_half), jnp.float32),
            pltpu.VMEM((2, h_half, d_half), jnp.float32),
            pltpu.VMEM((3, 2, b, h_half // 2), bf16),
            pltpu.VMEM((3, 2, b, h_half // 2), bf16),
            pltpu.VMEM((3, 2, b, d_half // 2), bf16),
            pltpu.VMEM((3, 2, b, d_half // 2), bf16),
            pltpu.SemaphoreType.DMA((2 * N_PARTS,)),
            pltpu.SemaphoreType.DMA((6,)),
            pltpu.SemaphoreType.DMA((6,)),
            pltpu.SemaphoreType.DMA((6,)),
            pltpu.SemaphoreType.DMA((6,)),
        ],
        compiler_params=pltpu.CompilerParams(
            collective_id=0, vmem_limit_bytes=100 * 1024 * 1024,
        ),
    )(x, Win0, Wout0, Win1, Wout1, Win2, Wout2)


# device time: 37554 ns/iter; 1.0255x vs baseline; 1.0250x over previous
import jax
import jax.numpy as jnp
from jax import lax
from jax.experimental import pallas as pl
from jax.experimental.pallas import tpu as pltpu

N_PARTS = 4


def kernel(x, Win0, Wout0, Win1, Wout1, Win2, Wout2):
    b, d_half = x.shape
    h_half = Win0.shape[1]
    bf16 = jnp.bfloat16

    def body(x_ref, win0_ref, wout0_ref, win1_ref, wout1_ref, win2_ref,
             wout2_ref, out_ref,
             win_buf, wout_buf, h_send, h_recv, x_send, x_recv,
             w_sems, h_send_sems, h_recv_sems, x_send_sems, x_recv_sems):
        my_x = lax.axis_index("x")
        my_y = lax.axis_index("y")
        y_peer = (my_x, 1 - my_y)
        x_peer = (1 - my_x, my_y)

        win_hbm = (win0_ref, win1_ref, win2_ref)
        wout_hbm = (wout0_ref, wout1_ref, wout2_ref)

        def w_parts(seq, hbm_ref, buf, slot, nrows):
            q = nrows // N_PARTS
            bank = (seq % 2) * N_PARTS
            return [
                pltpu.make_async_copy(
                    hbm_ref.at[pl.ds(c * q, q)],
                    buf.at[slot, pl.ds(c * q, q)],
                    w_sems.at[bank + c],
                )
                for c in range(N_PARTS)
            ]

        def win_parts(layer):
            return w_parts(2 * layer, win_hbm[layer], win_buf,
                           layer % 2, d_half)

        def wout_parts(layer):
            return w_parts(2 * layer + 1, wout_hbm[layer], wout_buf,
                           layer % 2, h_half)

        def start_all(parts):
            for p in parts:
                p.start()

        def wait_all(parts):
            for p in parts:
                p.wait()

        start_all(win_parts(0))

        barrier = pltpu.get_barrier_semaphore()
        for peer in (y_peer, x_peer):
            pl.semaphore_signal(
                barrier, inc=1, device_id=peer,
                device_id_type=pl.DeviceIdType.MESH,
            )
        pl.semaphore_wait(barrier, 2)

        def start_exchange(slot, partial_f32, send_buf, recv_buf, send_sems,
                           recv_sems, peer):
            send_buf[slot] = partial_f32.astype(bf16)
            rdma = pltpu.make_async_remote_copy(
                src_ref=send_buf.at[slot],
                dst_ref=recv_buf.at[slot],
                send_sem=send_sems.at[slot],
                recv_sem=recv_sems.at[slot],
                device_id=peer,
                device_id_type=pl.DeviceIdType.MESH,
            )
            rdma.start()
            return rdma

        acts = x_ref[...]
        wait_all(win_parts(0))
        start_all(wout_parts(0))
        for layer in range(3):
            hp = jnp.dot(acts, win_buf[layer % 2],
                         preferred_element_type=jnp.float32)
            rdma = start_exchange(layer, hp, h_send, h_recv, h_send_sems,
                                  h_recv_sems, y_peer)
            wait_all(wout_parts(layer))
            if layer < 2:
                start_all(win_parts(layer + 1))
            rdma.wait()
            h = hp + h_recv[layer].astype(jnp.float32)
            h = jnp.maximum(h, 0.0)
            xp = jnp.dot(h, wout_buf[layer % 2],
                         preferred_element_type=jnp.float32)
            rdma = start_exchange(layer, xp, x_send, x_recv, x_send_sems,
                                  x_recv_sems, x_peer)
            if layer < 2:
                wait_all(win_parts(layer + 1))
                start_all(wout_parts(layer + 1))
            rdma.wait()
            acts = xp + x_recv[layer].astype(jnp.float32)
        out_ref[...] = acts

    vmem = pl.BlockSpec(memory_space=pltpu.VMEM)
    hbm = pl.BlockSpec(memory_space=pl.ANY)
    return pl.pallas_call(
        body,
        out_shape=jax.ShapeDtypeStruct((b, d_half), jnp.float32),
        in_specs=[vmem] + [hbm] * 6,
        out_specs=vmem,
        scratch_shapes=[
            pltpu.VMEM((2, d_half, h_half), jnp.float32),
            pltpu.VMEM((2, h_half, d_half), jnp.float32),
            pltpu.VMEM((3, b, h_half), bf16),
            pltpu.VMEM((3, b, h_half), bf16),
            pltpu.VMEM((3, b, d_half), bf16),
            pltpu.VMEM((3, b, d_half), bf16),
            pltpu.SemaphoreType.DMA((2 * N_PARTS,)),
            pltpu.SemaphoreType.DMA((3,)),
            pltpu.SemaphoreType.DMA((3,)),
            pltpu.SemaphoreType.DMA((3,)),
            pltpu.SemaphoreType.DMA((3,)),
        ],
        compiler_params=pltpu.CompilerParams(
            collective_id=0, vmem_limit_bytes=100 * 1024 * 1024,
        ),
    )(x, Win0, Wout0, Win1, Wout1, Win2, Wout2)
